# baseline (speedup 1.0000x reference)
import functools

import jax
import jax.numpy as jnp
from jax import lax
from jax.experimental import pallas as pl
from jax.experimental.pallas import tpu as pltpu
from jax.experimental.pallas import tpu_sc as plsc

TOKENS = 32768
EMBED_DIM = 768
NUM_EXPERTS = 8
TOP_K = 2
BLOCK = 4096

_SC_INFO = plsc.get_sparse_core_info()
_NC = _SC_INFO.num_cores
_NS = _SC_INFO.num_subcores
_NW = _NC * _NS
_PER_W = TOKENS // _NW
_LANES = 16
_CHUNKS = _PER_W // _LANES


def _logits_kernel(x_ref, w_ref, logits_ref):
    x = x_ref[...]
    w = w_ref[...]
    logits = jax.lax.dot_general(
        x, w, (((1,), (1,)), ((), ())), preferred_element_type=jnp.float32
    )
    logits_ref[...] = logits.T


def _sc_top2_kernel(logits_hbm, idx_hbm, wgt_hbm, lg_v, idx_v, wgt_v):
    wid = lax.axis_index("s") * _NC + lax.axis_index("c")
    base = wid * _PER_W
    pltpu.sync_copy(logits_hbm.at[:, pl.ds(base, _PER_W)], lg_v)

    def body(ci, _):
        off = ci * _LANES
        rows = [lg_v[r, pl.ds(off, _LANES)] for r in range(NUM_EXPERTS)]
        m1 = rows[0]
        i1 = jnp.zeros((_LANES,), jnp.int32)
        for r in range(1, NUM_EXPERTS):
            gt = rows[r] > m1
            m1 = jnp.where(gt, rows[r], m1)
            i1 = jnp.where(gt, jnp.full((_LANES,), r, jnp.int32), i1)
        neg = jnp.full((_LANES,), -jnp.inf, jnp.float32)
        m2 = neg
        i2 = jnp.zeros((_LANES,), jnp.int32)
        for r in range(NUM_EXPERTS):
            rv = jnp.full((_LANES,), r, jnp.int32)
            cand = jnp.where(i1 == rv, neg, rows[r])
            gt = cand > m2
            m2 = jnp.where(gt, cand, m2)
            i2 = jnp.where(gt, rv, i2)
        w1 = 1.0 / (1.0 + jnp.exp(m2 - m1))
        w2 = 1.0 - w1
        idx_v[0, pl.ds(off, _LANES)] = i1
        idx_v[1, pl.ds(off, _LANES)] = i2
        wgt_v[0, pl.ds(off, _LANES)] = w1
        wgt_v[1, pl.ds(off, _LANES)] = w2
        return ()

    lax.fori_loop(0, _CHUNKS, body, ())

    pltpu.sync_copy(idx_v, idx_hbm.at[:, pl.ds(base, _PER_W)])
    pltpu.sync_copy(wgt_v, wgt_hbm.at[:, pl.ds(base, _PER_W)])


_sc_top2 = functools.partial(
    pl.kernel,
    mesh=plsc.VectorSubcoreMesh(core_axis_name="c", subcore_axis_name="s"),
    out_type=[
        jax.ShapeDtypeStruct((TOP_K, TOKENS), jnp.int32),
        jax.ShapeDtypeStruct((TOP_K, TOKENS), jnp.float32),
    ],
    scratch_types=[
        pltpu.VMEM((NUM_EXPERTS, _PER_W), jnp.float32),
        pltpu.VMEM((TOP_K, _PER_W), jnp.int32),
        pltpu.VMEM((TOP_K, _PER_W), jnp.float32),
    ],
)(_sc_top2_kernel)


@jax.jit
def kernel(x, W):
    nb = TOKENS // BLOCK
    logits_t = pl.pallas_call(
        _logits_kernel,
        grid=(nb,),
        in_specs=[
            pl.BlockSpec((BLOCK, EMBED_DIM), lambda i: (i, 0)),
            pl.BlockSpec((NUM_EXPERTS, EMBED_DIM), lambda i: (0, 0)),
        ],
        out_specs=[
            pl.BlockSpec((NUM_EXPERTS, BLOCK), lambda i: (0, i)),
        ],
        compiler_params=pltpu.CompilerParams(
            dimension_semantics=("arbitrary",),
        ),
        out_shape=[
            jax.ShapeDtypeStruct((NUM_EXPERTS, TOKENS), jnp.float32),
        ],
    )(x, W)[0]
    idx_t, wgt_t = _sc_top2(logits_t)
    return idx_t.T, wgt_t.T, logits_t.T


# speedup vs baseline: 1.5797x; 1.5797x over previous
import functools

import jax
import jax.numpy as jnp
from jax.experimental import pallas as pl
from jax.experimental.pallas import tpu as pltpu

TOKENS = 32768
EMBED_DIM = 768
NUM_EXPERTS = 8
TOP_K = 2
BLOCK = 4096


def _router_kernel(x_ref, w_ref, idx_ref, wgt_ref, logits_ref):
    x = x_ref[...]
    w = w_ref[...]
    logits = jax.lax.dot_general(
        x, w, (((1,), (1,)), ((), ())), preferred_element_type=jnp.float32
    )
    logits_t = logits.T
    logits_ref[...] = logits_t

    i1 = jnp.argmax(logits_t, axis=0)
    m1 = jnp.max(logits_t, axis=0)
    e = jax.lax.broadcasted_iota(jnp.int32, logits_t.shape, 0)
    masked = jnp.where(e == i1[None, :], -jnp.inf, logits_t)
    i2 = jnp.argmax(masked, axis=0)
    m2 = jnp.max(masked, axis=0)

    w1 = 1.0 / (1.0 + jnp.exp(m2 - m1))
    w2 = 1.0 - w1

    idx_ref[...] = jnp.concatenate(
        [i1[None, :].astype(jnp.int32), i2[None, :].astype(jnp.int32)], axis=0
    )
    wgt_ref[...] = jnp.concatenate([w1[None, :], w2[None, :]], axis=0)


@jax.jit
def kernel(x, W):
    nb = TOKENS // BLOCK
    idx_t, wgt_t, logits_t = pl.pallas_call(
        _router_kernel,
        grid=(nb,),
        in_specs=[
            pl.BlockSpec((BLOCK, EMBED_DIM), lambda i: (i, 0)),
            pl.BlockSpec((NUM_EXPERTS, EMBED_DIM), lambda i: (0, 0)),
        ],
        out_specs=[
            pl.BlockSpec((TOP_K, BLOCK), lambda i: (0, i)),
            pl.BlockSpec((TOP_K, BLOCK), lambda i: (0, i)),
            pl.BlockSpec((NUM_EXPERTS, BLOCK), lambda i: (0, i)),
        ],
        compiler_params=pltpu.CompilerParams(
            dimension_semantics=("arbitrary",),
        ),
        out_shape=[
            jax.ShapeDtypeStruct((TOP_K, TOKENS), jnp.int32),
            jax.ShapeDtypeStruct((TOP_K, TOKENS), jnp.float32),
            jax.ShapeDtypeStruct((NUM_EXPERTS, TOKENS), jnp.float32),
        ],
    )(x, W)
    return idx_t.T, wgt_t.T, logits_t.T
